# Initial kernel scaffold; baseline (speedup 1.0000x reference)
import jax
import jax.numpy as jnp
from jax.experimental import pallas as pl


def kernel(x, edge_index, batch, W1, b1, W2, b2, Wf, bf):
    raise NotImplementedError("write your pallas kernel here")


# speedup vs baseline: 8.2313x; 8.2313x over previous
import functools
import jax
import jax.numpy as jnp
from jax import lax
from jax.experimental import pallas as pl
from jax.experimental.pallas import tpu as pltpu
from jax.experimental.pallas import tpu_sc as plsc

N_NODES = 10000
N_EDGES = 320000
D = 128
N_GRAPHS = 64

NC = 2
NS = 16
NW = NC * NS
N_PAD = 10240
CH = 128
NCH = 80
EPT = NCH * CH
N_EPAD = NW * EPT
ROWS_PER_TILE = N_PAD // NS


def _sc_hist_body(dst_hbm, out_hbm, idx_v, ones_v, zero_v, hist_sp):
    c = lax.axis_index("c")
    s = lax.axis_index("s")
    wid = c * NS + s

    one16 = jnp.ones((16,), jnp.float32)
    zero16 = jnp.zeros((16,), jnp.float32)

    def fill_ones(i, _):
        ones_v[pl.ds(i * 16, 16)] = one16
        return 0

    lax.fori_loop(0, CH // 16, fill_ones, 0)

    def fill_zero(i, _):
        zero_v[pl.ds(i * 16, 16)] = zero16
        return 0

    lax.fori_loop(0, ROWS_PER_TILE // 16, fill_zero, 0)

    pltpu.sync_copy(dst_hbm.at[wid], idx_v)

    pltpu.sync_copy(zero_v, hist_sp.at[pl.ds(s * ROWS_PER_TILE, ROWS_PER_TILE)])
    plsc.subcore_barrier()

    def chunk(j, _):
        pltpu.sync_copy(ones_v, hist_sp.at[idx_v.at[j]], add=True)
        return 0

    lax.fori_loop(0, NCH, chunk, 0)
    plsc.subcore_barrier()

    pltpu.sync_copy(
        hist_sp.at[pl.ds(s * ROWS_PER_TILE, ROWS_PER_TILE)],
        out_hbm.at[c, pl.ds(s * ROWS_PER_TILE, ROWS_PER_TILE)],
    )


_sc_hist = pl.kernel(
    _sc_hist_body,
    out_type=jax.ShapeDtypeStruct((NC, N_PAD), jnp.float32),
    mesh=plsc.VectorSubcoreMesh(core_axis_name="c", subcore_axis_name="s"),
    scratch_types=[
        pltpu.VMEM((NCH, CH), jnp.int32),
        pltpu.VMEM((CH,), jnp.float32),
        pltpu.VMEM((ROWS_PER_TILE,), jnp.float32),
        pltpu.VMEM_SHARED((N_PAD,), jnp.float32),
    ],
)


def _sc_scatter_body(hs_hbm, src_hbm, dst_hbm, out_hbm,
                     srcv, dstv, rows_v, zrow_v, gsem, acc_sp):
    c = lax.axis_index("c")
    s = lax.axis_index("s")
    wid = c * NS + s

    zero16 = jnp.zeros((16,), jnp.float32)

    def zfill_row(i, _):
        def zfill_col(k, _):
            zrow_v[i, pl.ds(k * 16, 16)] = zero16
            return 0
        lax.fori_loop(0, D // 16, zfill_col, 0)
        return 0

    lax.fori_loop(0, 64, zfill_row, 0)

    pltpu.sync_copy(src_hbm.at[wid], srcv)
    pltpu.sync_copy(dst_hbm.at[wid], dstv)

    def zcopy(r, _):
        pltpu.sync_copy(
            zrow_v, acc_sp.at[pl.ds(s * ROWS_PER_TILE + r * 64, 64), :])
        return 0

    lax.fori_loop(0, ROWS_PER_TILE // 64, zcopy, 0)
    plsc.subcore_barrier()

    def chunk(j, _):
        pltpu.async_copy(hs_hbm.at[srcv.at[j]], rows_v, gsem).wait()
        pltpu.sync_copy(rows_v, acc_sp.at[dstv.at[j]], add=True)
        return 0

    lax.fori_loop(0, NCH, chunk, 0)
    plsc.subcore_barrier()

    pltpu.sync_copy(
        acc_sp.at[pl.ds(s * ROWS_PER_TILE, ROWS_PER_TILE), :],
        out_hbm.at[c, pl.ds(s * ROWS_PER_TILE, ROWS_PER_TILE), :],
    )


_sc_scatter = pl.kernel(
    _sc_scatter_body,
    out_type=jax.ShapeDtypeStruct((NC, N_PAD, D), jnp.float32),
    mesh=plsc.VectorSubcoreMesh(core_axis_name="c", subcore_axis_name="s"),
    scratch_types=[
        pltpu.VMEM((NCH, CH), jnp.int32),
        pltpu.VMEM((NCH, CH), jnp.int32),
        pltpu.VMEM((CH, D), jnp.float32),
        pltpu.VMEM((64, D), jnp.float32),
        pltpu.SemaphoreType.DMA,
        pltpu.VMEM_SHARED((N_PAD, D), jnp.float32),
    ],
)


BLK = 640


def _tc_lin1_body(x_ref, w_ref, h0_ref, h1_ref, hs_ref, dinv_ref):
    deg = h0_ref[...] + h1_ref[...] + 1.0
    dinv = lax.rsqrt(deg)
    h = jnp.dot(x_ref[...], w_ref[...], preferred_element_type=jnp.float32)
    hs_ref[...] = h * dinv
    dinv_ref[...] = dinv


def _tc_lin1(x, w1, h0, h1):
    return pl.pallas_call(
        _tc_lin1_body,
        grid=(N_PAD // BLK,),
        in_specs=[
            pl.BlockSpec((BLK, D), lambda i: (i, 0)),
            pl.BlockSpec((D, D), lambda i: (0, 0)),
            pl.BlockSpec((BLK, 1), lambda i: (i, 0)),
            pl.BlockSpec((BLK, 1), lambda i: (i, 0)),
        ],
        out_specs=[
            pl.BlockSpec((BLK, D), lambda i: (i, 0)),
            pl.BlockSpec((BLK, 1), lambda i: (i, 0)),
        ],
        out_shape=[
            jax.ShapeDtypeStruct((N_PAD, D), jnp.float32),
            jax.ShapeDtypeStruct((N_PAD, 1), jnp.float32),
        ],
    )(x, w1, h0, h1)


def _tc_mid_body(p0_ref, p1_ref, hs_ref, dinv_ref, b_ref, w_ref, out_ref):
    dinv = dinv_ref[...]
    t = (p0_ref[...] + p1_ref[...] + hs_ref[...]) * dinv + b_ref[...]
    r = jnp.maximum(t, 0.0)
    out_ref[...] = jnp.dot(
        r, w_ref[...], preferred_element_type=jnp.float32) * dinv


def _tc_mid(p0, p1, hs, dinv, b1, w2):
    return pl.pallas_call(
        _tc_mid_body,
        grid=(N_PAD // BLK,),
        in_specs=[
            pl.BlockSpec((BLK, D), lambda i: (i, 0)),
            pl.BlockSpec((BLK, D), lambda i: (i, 0)),
            pl.BlockSpec((BLK, D), lambda i: (i, 0)),
            pl.BlockSpec((BLK, 1), lambda i: (i, 0)),
            pl.BlockSpec((1, D), lambda i: (0, 0)),
            pl.BlockSpec((D, D), lambda i: (0, 0)),
        ],
        out_specs=pl.BlockSpec((BLK, D), lambda i: (i, 0)),
        out_shape=jax.ShapeDtypeStruct((N_PAD, D), jnp.float32),
    )(p0, p1, hs, dinv, b1, w2)


def _tc_head_body(q0_ref, q1_ref, hs_ref, dinv_ref, b_ref, batch_ref,
                  wf_ref, bf_ref, out_ref, sums_ref, cnts_ref):
    i = pl.program_id(0)

    @pl.when(i == 0)
    def _():
        sums_ref[...] = jnp.zeros_like(sums_ref)
        cnts_ref[...] = jnp.zeros_like(cnts_ref)

    t = (q0_ref[...] + q1_ref[...] + hs_ref[...]) * dinv_ref[...] + b_ref[...]
    r = jnp.maximum(t, 0.0)

    gids = lax.broadcasted_iota(jnp.int32, (N_GRAPHS, BLK), 0)
    brow = batch_ref[...].reshape(1, BLK)
    mask = (gids == brow).astype(jnp.float32)
    sums_ref[...] += jnp.dot(mask, r, preferred_element_type=jnp.float32)
    cnts_ref[...] += jnp.broadcast_to(
        jnp.sum(mask, axis=1, keepdims=True), (N_GRAPHS, D))

    @pl.when(i == (N_PAD // BLK) - 1)
    def _():
        pooled = sums_ref[...] / jnp.maximum(cnts_ref[...], 1.0)
        out_ref[...] = jnp.dot(
            pooled, wf_ref[...], preferred_element_type=jnp.float32
        ) + bf_ref[...]


def _tc_head(q0, q1, hs2, dinv, b2, batch_rows, wf, bf):
    return pl.pallas_call(
        _tc_head_body,
        grid=(N_PAD // BLK,),
        in_specs=[
            pl.BlockSpec((BLK, D), lambda i: (i, 0)),
            pl.BlockSpec((BLK, D), lambda i: (i, 0)),
            pl.BlockSpec((BLK, D), lambda i: (i, 0)),
            pl.BlockSpec((BLK, 1), lambda i: (i, 0)),
            pl.BlockSpec((1, D), lambda i: (0, 0)),
            pl.BlockSpec((1, 1, BLK), lambda i: (i, 0, 0)),
            pl.BlockSpec((D, 1), lambda i: (0, 0)),
            pl.BlockSpec((1, 1), lambda i: (0, 0)),
        ],
        out_specs=pl.BlockSpec((N_GRAPHS, 1), lambda i: (0, 0)),
        out_shape=jax.ShapeDtypeStruct((N_GRAPHS, 1), jnp.float32),
        scratch_shapes=[
            pltpu.VMEM((N_GRAPHS, D), jnp.float32),
            pltpu.VMEM((N_GRAPHS, D), jnp.float32),
        ],
    )(q0, q1, hs2, dinv, b2, batch_rows, wf, bf)


@jax.jit
def kernel(x, edge_index, batch, W1, b1, W2, b2, Wf, bf):
    f32 = jnp.float32
    i32 = jnp.int32

    src = edge_index[0].astype(i32)
    dst = edge_index[1].astype(i32)
    n_fill = N_EPAD - N_EDGES
    src_p = jnp.concatenate([src, jnp.zeros((n_fill,), i32)])
    dst_p = jnp.concatenate([dst, jnp.full((n_fill,), N_PAD - 1, i32)])
    src3 = src_p.reshape(NW, NCH, CH)
    dst3 = dst_p.reshape(NW, NCH, CH)

    x_p = jnp.concatenate(
        [x.astype(f32), jnp.zeros((N_PAD - N_NODES, D), f32)])
    batch_p = jnp.concatenate(
        [batch.astype(i32), jnp.full((N_PAD - N_NODES,), N_GRAPHS, i32)])
    batch_rows = batch_p.reshape(N_PAD // BLK, 1, BLK)

    hist = _sc_hist(dst3)
    h0 = hist[0].reshape(N_PAD, 1)
    h1 = hist[1].reshape(N_PAD, 1)

    hs1, dinv = _tc_lin1(x_p, W1, h0, h1)

    p = _sc_scatter(hs1, src3, dst3)
    hs2 = _tc_mid(p[0], p[1], hs1, dinv, b1.reshape(1, D), W2)

    q = _sc_scatter(hs2, src3, dst3)
    out = _tc_head(q[0], q[1], hs2, dinv, b2.reshape(1, D), batch_rows,
                   Wf, bf.reshape(1, 1))
    return out


# speedup vs baseline: 9.5401x; 1.1590x over previous
import functools
import jax
import jax.numpy as jnp
from jax import lax
from jax.experimental import pallas as pl
from jax.experimental.pallas import tpu as pltpu
from jax.experimental.pallas import tpu_sc as plsc

N_NODES = 10000
N_EDGES = 320000
D = 128
N_GRAPHS = 64

NC = 2
NS = 16
NW = NC * NS
N_PAD = 10240
CH = 80
NCH = 128
EPT = NCH * CH
N_EPAD = NW * EPT
ROWS_PER_TILE = N_PAD // NS


def _sc_hist_body(dst_hbm, out_hbm, idx_v, ones_v, zero_v, hist_sp):
    c = lax.axis_index("c")
    s = lax.axis_index("s")
    wid = c * NS + s

    one16 = jnp.ones((16,), jnp.float32)
    zero16 = jnp.zeros((16,), jnp.float32)

    def fill_ones(i, _):
        ones_v[pl.ds(i * 16, 16)] = one16
        return 0

    lax.fori_loop(0, CH // 16, fill_ones, 0)

    def fill_zero(i, _):
        zero_v[pl.ds(i * 16, 16)] = zero16
        return 0

    lax.fori_loop(0, ROWS_PER_TILE // 16, fill_zero, 0)

    pltpu.sync_copy(dst_hbm.at[wid], idx_v)

    pltpu.sync_copy(zero_v, hist_sp.at[pl.ds(s * ROWS_PER_TILE, ROWS_PER_TILE)])
    plsc.subcore_barrier()

    def chunk(j, _):
        pltpu.sync_copy(ones_v, hist_sp.at[idx_v.at[j]], add=True)
        return 0

    lax.fori_loop(0, NCH, chunk, 0)
    plsc.subcore_barrier()

    pltpu.sync_copy(
        hist_sp.at[pl.ds(s * ROWS_PER_TILE, ROWS_PER_TILE)],
        out_hbm.at[c, pl.ds(s * ROWS_PER_TILE, ROWS_PER_TILE)],
    )


_sc_hist = pl.kernel(
    _sc_hist_body,
    out_type=jax.ShapeDtypeStruct((NC, N_PAD), jnp.float32),
    mesh=plsc.VectorSubcoreMesh(core_axis_name="c", subcore_axis_name="s"),
    scratch_types=[
        pltpu.VMEM((NCH, CH), jnp.int32),
        pltpu.VMEM((CH,), jnp.float32),
        pltpu.VMEM((ROWS_PER_TILE,), jnp.float32),
        pltpu.VMEM_SHARED((N_PAD,), jnp.float32),
    ],
)


NB = 2
NQ = 4
NCHQ = NCH // NQ


def _sc_scatter_body(hs_hbm, src_hbm, dst_hbm, out_hbm,
                     srcv, dstv, rows_v, zrow_v, gsems, acc_sp):
    c = lax.axis_index("c")
    s = lax.axis_index("s")
    wid = c * NS + s

    zero16 = jnp.zeros((16,), jnp.float32)

    def zfill_row(i, _):
        def zfill_col(k, _):
            zrow_v[i, pl.ds(k * 16, 16)] = zero16
            return 0
        lax.fori_loop(0, D // 16, zfill_col, 0)
        return 0

    lax.fori_loop(0, 16, zfill_row, 0)

    def zcopy(r, _):
        pltpu.sync_copy(
            zrow_v, acc_sp.at[pl.ds(s * ROWS_PER_TILE + r * 16, 16), :])
        return 0

    lax.fori_loop(0, ROWS_PER_TILE // 16, zcopy, 0)
    plsc.subcore_barrier()

    for q in range(NQ):
        pltpu.sync_copy(src_hbm.at[wid, pl.ds(q * NCHQ, NCHQ)], srcv)
        pltpu.sync_copy(dst_hbm.at[wid, pl.ds(q * NCHQ, NCHQ)], dstv)
        for b in range(NB):
            pltpu.async_copy(hs_hbm.at[srcv.at[b]], rows_v.at[b], gsems[b])

        def round_(k, _):
            for b in range(NB):
                j = k * NB + b
                pltpu.make_async_copy(
                    hs_hbm.at[srcv.at[j]], rows_v.at[b], gsems[b]).wait()
                pltpu.sync_copy(
                    rows_v.at[b], acc_sp.at[dstv.at[j]], add=True)

                @pl.when(j + NB < NCHQ)
                def _():
                    pltpu.async_copy(
                        hs_hbm.at[srcv.at[j + NB]], rows_v.at[b], gsems[b])
            return 0

        lax.fori_loop(0, NCHQ // NB, round_, 0)
    plsc.subcore_barrier()

    pltpu.sync_copy(
        acc_sp.at[pl.ds(s * ROWS_PER_TILE, ROWS_PER_TILE), :],
        out_hbm.at[c, pl.ds(s * ROWS_PER_TILE, ROWS_PER_TILE), :],
    )


_sc_scatter = pl.kernel(
    _sc_scatter_body,
    out_type=jax.ShapeDtypeStruct((NC, N_PAD, D), jnp.float32),
    mesh=plsc.VectorSubcoreMesh(core_axis_name="c", subcore_axis_name="s"),
    scratch_types=[
        pltpu.VMEM((NCHQ, CH), jnp.int32),
        pltpu.VMEM((NCHQ, CH), jnp.int32),
        pltpu.VMEM((NB, CH, D), jnp.float32),
        pltpu.VMEM((16, D), jnp.float32),
        [pltpu.SemaphoreType.DMA] * NB,
        pltpu.VMEM_SHARED((N_PAD, D), jnp.float32),
    ],
)


BLK = 640


def _tc_lin1_body(x_ref, w_ref, h0_ref, h1_ref, hs_ref, dinv_ref):
    deg = h0_ref[...] + h1_ref[...] + 1.0
    dinv = lax.rsqrt(deg)
    h = jnp.dot(x_ref[...], w_ref[...], preferred_element_type=jnp.float32)
    hs_ref[...] = h * dinv
    dinv_ref[...] = dinv


def _tc_lin1(x, w1, h0, h1):
    return pl.pallas_call(
        _tc_lin1_body,
        grid=(N_PAD // BLK,),
        in_specs=[
            pl.BlockSpec((BLK, D), lambda i: (i, 0)),
            pl.BlockSpec((D, D), lambda i: (0, 0)),
            pl.BlockSpec((BLK, 1), lambda i: (i, 0)),
            pl.BlockSpec((BLK, 1), lambda i: (i, 0)),
        ],
        out_specs=[
            pl.BlockSpec((BLK, D), lambda i: (i, 0)),
            pl.BlockSpec((BLK, 1), lambda i: (i, 0)),
        ],
        out_shape=[
            jax.ShapeDtypeStruct((N_PAD, D), jnp.float32),
            jax.ShapeDtypeStruct((N_PAD, 1), jnp.float32),
        ],
    )(x, w1, h0, h1)


def _tc_mid_body(p0_ref, p1_ref, hs_ref, dinv_ref, b_ref, w_ref, out_ref):
    dinv = dinv_ref[...]
    t = (p0_ref[...] + p1_ref[...] + hs_ref[...]) * dinv + b_ref[...]
    r = jnp.maximum(t, 0.0)
    out_ref[...] = jnp.dot(
        r, w_ref[...], preferred_element_type=jnp.float32) * dinv


def _tc_mid(p0, p1, hs, dinv, b1, w2):
    return pl.pallas_call(
        _tc_mid_body,
        grid=(N_PAD // BLK,),
        in_specs=[
            pl.BlockSpec((BLK, D), lambda i: (i, 0)),
            pl.BlockSpec((BLK, D), lambda i: (i, 0)),
            pl.BlockSpec((BLK, D), lambda i: (i, 0)),
            pl.BlockSpec((BLK, 1), lambda i: (i, 0)),
            pl.BlockSpec((1, D), lambda i: (0, 0)),
            pl.BlockSpec((D, D), lambda i: (0, 0)),
        ],
        out_specs=pl.BlockSpec((BLK, D), lambda i: (i, 0)),
        out_shape=jax.ShapeDtypeStruct((N_PAD, D), jnp.float32),
    )(p0, p1, hs, dinv, b1, w2)


def _tc_head_body(q0_ref, q1_ref, hs_ref, dinv_ref, b_ref, batch_ref,
                  wf_ref, bf_ref, out_ref, sums_ref, cnts_ref):
    i = pl.program_id(0)

    @pl.when(i == 0)
    def _():
        sums_ref[...] = jnp.zeros_like(sums_ref)
        cnts_ref[...] = jnp.zeros_like(cnts_ref)

    t = (q0_ref[...] + q1_ref[...] + hs_ref[...]) * dinv_ref[...] + b_ref[...]
    r = jnp.maximum(t, 0.0)

    gids = lax.broadcasted_iota(jnp.int32, (N_GRAPHS, BLK), 0)
    brow = batch_ref[...].reshape(1, BLK)
    mask = (gids == brow).astype(jnp.float32)
    sums_ref[...] += jnp.dot(mask, r, preferred_element_type=jnp.float32,
                             precision=lax.Precision.HIGHEST)
    cnts_ref[...] += jnp.broadcast_to(
        jnp.sum(mask, axis=1, keepdims=True), (N_GRAPHS, D))

    @pl.when(i == (N_PAD // BLK) - 1)
    def _():
        pooled = sums_ref[...] / jnp.maximum(cnts_ref[...], 1.0)
        out_ref[...] = jnp.dot(
            pooled, wf_ref[...], preferred_element_type=jnp.float32) + bf_ref[...]


def _tc_head(q0, q1, hs2, dinv, b2, batch_rows, wf, bf):
    return pl.pallas_call(
        _tc_head_body,
        grid=(N_PAD // BLK,),
        in_specs=[
            pl.BlockSpec((BLK, D), lambda i: (i, 0)),
            pl.BlockSpec((BLK, D), lambda i: (i, 0)),
            pl.BlockSpec((BLK, D), lambda i: (i, 0)),
            pl.BlockSpec((BLK, 1), lambda i: (i, 0)),
            pl.BlockSpec((1, D), lambda i: (0, 0)),
            pl.BlockSpec((1, 1, BLK), lambda i: (i, 0, 0)),
            pl.BlockSpec((D, 1), lambda i: (0, 0)),
            pl.BlockSpec((1, 1), lambda i: (0, 0)),
        ],
        out_specs=pl.BlockSpec((N_GRAPHS, 1), lambda i: (0, 0)),
        out_shape=jax.ShapeDtypeStruct((N_GRAPHS, 1), jnp.float32),
        scratch_shapes=[
            pltpu.VMEM((N_GRAPHS, D), jnp.float32),
            pltpu.VMEM((N_GRAPHS, D), jnp.float32),
        ],
    )(q0, q1, hs2, dinv, b2, batch_rows, wf, bf)


@jax.jit
def kernel(x, edge_index, batch, W1, b1, W2, b2, Wf, bf):
    f32 = jnp.float32
    i32 = jnp.int32

    src = edge_index[0].astype(i32)
    dst = edge_index[1].astype(i32)
    n_fill = N_EPAD - N_EDGES
    src_p = jnp.concatenate([src, jnp.zeros((n_fill,), i32)])
    dst_p = jnp.concatenate([dst, jnp.full((n_fill,), N_PAD - 1, i32)])
    src3 = src_p.reshape(NW, NCH, CH)
    dst3 = dst_p.reshape(NW, NCH, CH)

    x_p = jnp.concatenate(
        [x.astype(f32), jnp.zeros((N_PAD - N_NODES, D), f32)])
    batch_p = jnp.concatenate(
        [batch.astype(i32), jnp.full((N_PAD - N_NODES,), N_GRAPHS, i32)])
    batch_rows = batch_p.reshape(N_PAD // BLK, 1, BLK)

    hist = _sc_hist(dst3)
    h0 = hist[0].reshape(N_PAD, 1)
    h1 = hist[1].reshape(N_PAD, 1)

    hs1, dinv = _tc_lin1(x_p, W1, h0, h1)

    p = _sc_scatter(hs1, src3, dst3)
    hs2 = _tc_mid(p[0], p[1], hs1, dinv, b1.reshape(1, D), W2)

    q = _sc_scatter(hs2, src3, dst3)
    out = _tc_head(q[0], q[1], hs2, dinv, b2.reshape(1, D), batch_rows,
                   Wf, bf.reshape(1, 1))
    return out


# speedup vs baseline: 10.6984x; 1.1214x over previous
import functools
import jax
import jax.numpy as jnp
from jax import lax
from jax.experimental import pallas as pl
from jax.experimental.pallas import tpu as pltpu
from jax.experimental.pallas import tpu_sc as plsc

N_NODES = 10000
N_EDGES = 320000
D = 128
N_GRAPHS = 64

NC = 2
NS = 16
NW = NC * NS
N_PAD = 10240
CH = 80
NCH = 128
EPT = NCH * CH
N_EPAD = NW * EPT
ROWS_PER_TILE = N_PAD // NS


def _sc_hist_body(dst_hbm, out_hbm, idx_v, ones_v, zero_v, hist_sp):
    c = lax.axis_index("c")
    s = lax.axis_index("s")
    wid = c * NS + s

    one16 = jnp.ones((16,), jnp.float32)
    zero16 = jnp.zeros((16,), jnp.float32)

    def fill_ones(i, _):
        ones_v[pl.ds(i * 16, 16)] = one16
        return 0

    lax.fori_loop(0, CH // 16, fill_ones, 0)

    def fill_zero(i, _):
        zero_v[pl.ds(i * 16, 16)] = zero16
        return 0

    lax.fori_loop(0, ROWS_PER_TILE // 16, fill_zero, 0)

    pltpu.sync_copy(dst_hbm.at[wid], idx_v)

    pltpu.sync_copy(zero_v, hist_sp.at[pl.ds(s * ROWS_PER_TILE, ROWS_PER_TILE)])
    plsc.subcore_barrier()

    def chunk(j, _):
        pltpu.sync_copy(ones_v, hist_sp.at[idx_v.at[j]], add=True)
        return 0

    lax.fori_loop(0, NCH, chunk, 0)
    plsc.subcore_barrier()

    pltpu.sync_copy(
        hist_sp.at[pl.ds(s * ROWS_PER_TILE, ROWS_PER_TILE)],
        out_hbm.at[c, pl.ds(s * ROWS_PER_TILE, ROWS_PER_TILE)],
    )


_sc_hist = pl.kernel(
    _sc_hist_body,
    out_type=jax.ShapeDtypeStruct((NC, N_PAD), jnp.float32),
    mesh=plsc.VectorSubcoreMesh(core_axis_name="c", subcore_axis_name="s"),
    scratch_types=[
        pltpu.VMEM((NCH, CH), jnp.int32),
        pltpu.VMEM((CH,), jnp.float32),
        pltpu.VMEM((ROWS_PER_TILE,), jnp.float32),
        pltpu.VMEM_SHARED((N_PAD,), jnp.float32),
    ],
)


NB = 2
NQ = 4
NCHQ = NCH // NQ


def _sc_scatter_body(hs_hbm, src_hbm, dst_hbm, out_hbm,
                     srcv, dstv, rows_v, zrow_v, gsems, acc_sp):
    c = lax.axis_index("c")
    s = lax.axis_index("s")
    wid = c * NS + s

    zero16 = jnp.zeros((16,), jnp.float32)

    def zfill_row(i, _):
        def zfill_col(k, _):
            zrow_v[i, pl.ds(k * 16, 16)] = zero16
            return 0
        lax.fori_loop(0, D // 16, zfill_col, 0)
        return 0

    lax.fori_loop(0, 16, zfill_row, 0)

    def zcopy(r, _):
        pltpu.sync_copy(
            zrow_v, acc_sp.at[pl.ds(s * ROWS_PER_TILE + r * 16, 16), :])
        return 0

    lax.fori_loop(0, ROWS_PER_TILE // 16, zcopy, 0)
    plsc.subcore_barrier()

    for q in range(NQ):
        pltpu.sync_copy(src_hbm.at[wid, pl.ds(q * NCHQ, NCHQ)], srcv)
        pltpu.sync_copy(dst_hbm.at[wid, pl.ds(q * NCHQ, NCHQ)], dstv)
        for b in range(NB):
            pltpu.async_copy(hs_hbm.at[srcv.at[b]], rows_v.at[b], gsems[b])

        def round_(k, _):
            for b in range(NB):
                j = k * NB + b
                pltpu.make_async_copy(
                    hs_hbm.at[srcv.at[j]], rows_v.at[b], gsems[b]).wait()
                pltpu.sync_copy(
                    rows_v.at[b], acc_sp.at[dstv.at[j]], add=True)

                @pl.when(j + NB < NCHQ)
                def _():
                    pltpu.async_copy(
                        hs_hbm.at[srcv.at[j + NB]], rows_v.at[b], gsems[b])
            return 0

        lax.fori_loop(0, NCHQ // NB, round_, 0)
    plsc.subcore_barrier()

    pltpu.sync_copy(
        acc_sp.at[pl.ds(s * ROWS_PER_TILE, ROWS_PER_TILE), :],
        out_hbm.at[c, pl.ds(s * ROWS_PER_TILE, ROWS_PER_TILE), :],
    )


_sc_scatter = pl.kernel(
    _sc_scatter_body,
    out_type=jax.ShapeDtypeStruct((NC, N_PAD, D), jnp.float32),
    mesh=plsc.VectorSubcoreMesh(core_axis_name="c", subcore_axis_name="s"),
    scratch_types=[
        pltpu.VMEM((NCHQ, CH), jnp.int32),
        pltpu.VMEM((NCHQ, CH), jnp.int32),
        pltpu.VMEM((NB, CH, D), jnp.float32),
        pltpu.VMEM((16, D), jnp.float32),
        [pltpu.SemaphoreType.DMA] * NB,
        pltpu.VMEM_SHARED((N_PAD, D), jnp.float32),
    ],
)


BLK = 640


def _tc_lin1_body(x_ref, w_ref, h0_ref, h1_ref, hs_ref, dinv_ref):
    deg = h0_ref[...] + h1_ref[...] + 1.0
    dinv = lax.rsqrt(deg)
    h = jnp.dot(x_ref[...], w_ref[...], preferred_element_type=jnp.float32)
    hs_ref[...] = h * dinv
    dinv_ref[...] = dinv


def _tc_lin1(x, w1, h0, h1):
    return pl.pallas_call(
        _tc_lin1_body,
        grid=(N_PAD // BLK,),
        in_specs=[
            pl.BlockSpec((BLK, D), lambda i: (i, 0)),
            pl.BlockSpec((D, D), lambda i: (0, 0)),
            pl.BlockSpec((BLK, 1), lambda i: (i, 0)),
            pl.BlockSpec((BLK, 1), lambda i: (i, 0)),
        ],
        out_specs=[
            pl.BlockSpec((BLK, D), lambda i: (i, 0)),
            pl.BlockSpec((BLK, 1), lambda i: (i, 0)),
        ],
        out_shape=[
            jax.ShapeDtypeStruct((N_PAD, D), jnp.float32),
            jax.ShapeDtypeStruct((N_PAD, 1), jnp.float32),
        ],
    )(x, w1, h0, h1)


def _tc_mid_body(p0_ref, p1_ref, hs_ref, dinv_ref, b_ref, w_ref, out_ref):
    dinv = dinv_ref[...]
    t = (p0_ref[...] + p1_ref[...] + hs_ref[...]) * dinv + b_ref[...]
    r = jnp.maximum(t, 0.0)
    out_ref[...] = jnp.dot(
        r, w_ref[...], preferred_element_type=jnp.float32) * dinv


def _tc_mid(p0, p1, hs, dinv, b1, w2):
    return pl.pallas_call(
        _tc_mid_body,
        grid=(N_PAD // BLK,),
        in_specs=[
            pl.BlockSpec((BLK, D), lambda i: (i, 0)),
            pl.BlockSpec((BLK, D), lambda i: (i, 0)),
            pl.BlockSpec((BLK, D), lambda i: (i, 0)),
            pl.BlockSpec((BLK, 1), lambda i: (i, 0)),
            pl.BlockSpec((1, D), lambda i: (0, 0)),
            pl.BlockSpec((D, D), lambda i: (0, 0)),
        ],
        out_specs=pl.BlockSpec((BLK, D), lambda i: (i, 0)),
        out_shape=jax.ShapeDtypeStruct((N_PAD, D), jnp.float32),
    )(p0, p1, hs, dinv, b1, w2)


def _tc_head_body(q0_ref, q1_ref, hs_ref, dinv_ref, b_ref, batch_ref,
                  wf_ref, bf_ref, out_ref, sums_ref, cnts_ref):
    i = pl.program_id(0)

    @pl.when(i == 0)
    def _():
        sums_ref[...] = jnp.zeros_like(sums_ref)
        cnts_ref[...] = jnp.zeros_like(cnts_ref)

    t = (q0_ref[...] + q1_ref[...] + hs_ref[...]) * dinv_ref[...] + b_ref[...]
    r = jnp.maximum(t, 0.0)

    gids = lax.broadcasted_iota(jnp.int32, (N_GRAPHS, BLK), 0)
    brow = batch_ref[...].reshape(1, BLK)
    mask = (gids == brow).astype(jnp.float32)
    sums_ref[...] += jnp.dot(mask, r, preferred_element_type=jnp.float32,
                             precision=lax.Precision.HIGHEST)
    cnts_ref[...] += jnp.broadcast_to(
        jnp.sum(mask, axis=1, keepdims=True), (N_GRAPHS, D))

    @pl.when(i == (N_PAD // BLK) - 1)
    def _():
        pooled = sums_ref[...] / jnp.maximum(cnts_ref[...], 1.0)
        out_ref[...] = jnp.dot(
            pooled, wf_ref[...], preferred_element_type=jnp.float32) + bf_ref[...]


def _tc_head(q0, q1, hs2, dinv, b2, batch_rows, wf, bf):
    return pl.pallas_call(
        _tc_head_body,
        grid=(N_PAD // BLK,),
        in_specs=[
            pl.BlockSpec((BLK, D), lambda i: (i, 0)),
            pl.BlockSpec((BLK, D), lambda i: (i, 0)),
            pl.BlockSpec((BLK, D), lambda i: (i, 0)),
            pl.BlockSpec((BLK, 1), lambda i: (i, 0)),
            pl.BlockSpec((1, D), lambda i: (0, 0)),
            pl.BlockSpec((1, 1, BLK), lambda i: (i, 0, 0)),
            pl.BlockSpec((D, 1), lambda i: (0, 0)),
            pl.BlockSpec((1, 1), lambda i: (0, 0)),
        ],
        out_specs=pl.BlockSpec((N_GRAPHS, 1), lambda i: (0, 0)),
        out_shape=jax.ShapeDtypeStruct((N_GRAPHS, 1), jnp.float32),
        scratch_shapes=[
            pltpu.VMEM((N_GRAPHS, D), jnp.float32),
            pltpu.VMEM((N_GRAPHS, D), jnp.float32),
        ],
    )(q0, q1, hs2, dinv, b2, batch_rows, wf, bf)


@jax.jit
def kernel(x, edge_index, batch, W1, b1, W2, b2, Wf, bf):
    f32 = jnp.float32
    i32 = jnp.int32

    src = edge_index[0].astype(i32)
    dst = edge_index[1].astype(i32)
    ppt = EPT - N_EDGES // NW
    src2 = src.reshape(NW, N_EDGES // NW)
    dst2 = dst.reshape(NW, N_EDGES // NW)
    w = jnp.arange(NW, dtype=i32)[:, None]
    pad_src = jnp.zeros((NW, ppt), i32)
    pad_dst = (N_NODES + (w % NS) * (ppt // NS)
               + (jnp.arange(ppt, dtype=i32)[None, :] % (ppt // NS)))
    src3 = jnp.concatenate([src2, pad_src], axis=1).reshape(NW, NCH, CH)
    dst3 = jnp.concatenate([dst2, pad_dst], axis=1).reshape(NW, NCH, CH)

    x_p = jnp.concatenate(
        [x.astype(f32), jnp.zeros((N_PAD - N_NODES, D), f32)])
    batch_p = jnp.concatenate(
        [batch.astype(i32), jnp.full((N_PAD - N_NODES,), N_GRAPHS, i32)])
    batch_rows = batch_p.reshape(N_PAD // BLK, 1, BLK)

    hist = _sc_hist(dst3)
    h0 = hist[0].reshape(N_PAD, 1)
    h1 = hist[1].reshape(N_PAD, 1)

    hs1, dinv = _tc_lin1(x_p, W1, h0, h1)

    p = _sc_scatter(hs1, src3, dst3)
    hs2 = _tc_mid(p[0], p[1], hs1, dinv, b1.reshape(1, D), W2)

    q = _sc_scatter(hs2, src3, dst3)
    out = _tc_head(q[0], q[1], hs2, dinv, b2.reshape(1, D), batch_rows,
                   Wf, bf.reshape(1, 1))
    return out


# speedup vs baseline: 31.8569x; 2.9777x over previous
import jax
import jax.numpy as jnp
from jax import lax
from jax.experimental import pallas as pl
from jax.experimental.pallas import tpu as pltpu
from jax.experimental.pallas import tpu_sc as plsc

N_NODES = 10000
N_EDGES = 320000
D = 128
N_GRAPHS = 64

NC = 2
NS = 16
NW = NC * NS
EPT = N_EDGES // NW
CH = 80
NCH = EPT // CH
N_ACC = 10240
RPT = N_ACC // NS
LAST_ROWS = N_NODES - 15 * RPT
NB = 3
NQ = 5
NCHQ = NCH // NQ
NFULL = (NCHQ // NB) * NB


def _sc_hist_body(dst_hbm, out_hbm, idx_v, ones_v, zero_v, hist_sp):
    c = lax.axis_index("c")
    s = lax.axis_index("s")
    wid = c * NS + s

    one16 = jnp.ones((16,), jnp.float32)
    zero16 = jnp.zeros((16,), jnp.float32)

    def fill_ones(i, _):
        ones_v[pl.ds(i * 16, 16)] = one16
        return 0

    lax.fori_loop(0, CH // 16, fill_ones, 0)

    def fill_zero(i, _):
        zero_v[pl.ds(i * 16, 16)] = zero16
        return 0

    lax.fori_loop(0, RPT // 16, fill_zero, 0)

    pltpu.sync_copy(dst_hbm.at[wid], idx_v)

    pltpu.sync_copy(zero_v, hist_sp.at[pl.ds(s * RPT, RPT)])
    plsc.subcore_barrier()

    def chunk(j, _):
        pltpu.sync_copy(
            ones_v, hist_sp.at[idx_v.at[j // NCHQ, j % NCHQ]], add=True)
        return 0

    lax.fori_loop(0, NCH, chunk, 0)
    plsc.subcore_barrier()

    pltpu.sync_copy(
        hist_sp.at[pl.ds(s * RPT, RPT)],
        out_hbm.at[c, pl.ds(s * RPT, RPT)],
    )


_sc_hist = pl.kernel(
    _sc_hist_body,
    out_type=jax.ShapeDtypeStruct((NC, N_ACC), jnp.float32),
    mesh=plsc.VectorSubcoreMesh(core_axis_name="c", subcore_axis_name="s"),
    scratch_types=[
        pltpu.VMEM((NQ, NCHQ, CH), jnp.int32),
        pltpu.VMEM((CH,), jnp.float32),
        pltpu.VMEM((RPT,), jnp.float32),
        pltpu.VMEM_SHARED((N_ACC,), jnp.float32),
    ],
)


def _sc_scatter_body(hs_hbm, src_hbm, dst_hbm, out_hbm,
                     srcv, dstv, rows_v, zrow_v, gsems, acc_sp):
    c = lax.axis_index("c")
    s = lax.axis_index("s")
    wid = c * NS + s

    zero16 = jnp.zeros((16,), jnp.float32)

    def zfill_row(i, _):
        def zfill_col(k, _):
            zrow_v[i, pl.ds(k * 16, 16)] = zero16
            return 0
        lax.fori_loop(0, D // 16, zfill_col, 0)
        return 0

    lax.fori_loop(0, 8, zfill_row, 0)

    def zcopy(r, _):
        pltpu.sync_copy(zrow_v, acc_sp.at[pl.ds(s * RPT + r * 8, 8), :])
        return 0

    lax.fori_loop(0, RPT // 8, zcopy, 0)
    plsc.subcore_barrier()

    for q in range(NQ):
        pltpu.sync_copy(src_hbm.at[wid, q], srcv)
        pltpu.sync_copy(dst_hbm.at[wid, q], dstv)
        for b in range(NB):
            pltpu.async_copy(hs_hbm.at[srcv.at[b]], rows_v.at[b], gsems[b])

        def round_(k, _):
            for b in range(NB):
                j = k * NB + b
                pltpu.make_async_copy(
                    hs_hbm.at[srcv.at[j]], rows_v.at[b], gsems[b]).wait()
                pltpu.sync_copy(
                    rows_v.at[b], acc_sp.at[dstv.at[j]], add=True)

                @pl.when(j + NB < NCHQ)
                def _():
                    pltpu.async_copy(
                        hs_hbm.at[srcv.at[j + NB]], rows_v.at[b], gsems[b])
            return 0

        lax.fori_loop(0, NFULL // NB, round_, 0)
        for j in range(NFULL, NCHQ):
            b = j % NB
            pltpu.make_async_copy(
                hs_hbm.at[srcv.at[j]], rows_v.at[b], gsems[b]).wait()
            pltpu.sync_copy(rows_v.at[b], acc_sp.at[dstv.at[j]], add=True)
    plsc.subcore_barrier()

    @pl.when(s < NS - 1)
    def _():
        pltpu.sync_copy(
            acc_sp.at[pl.ds(s * RPT, RPT), :],
            out_hbm.at[c, pl.ds(s * RPT, RPT), :],
        )

    @pl.when(s == NS - 1)
    def _():
        pltpu.sync_copy(
            acc_sp.at[pl.ds((NS - 1) * RPT, LAST_ROWS), :],
            out_hbm.at[c, pl.ds((NS - 1) * RPT, LAST_ROWS), :],
        )


_sc_scatter = pl.kernel(
    _sc_scatter_body,
    out_type=jax.ShapeDtypeStruct((NC, N_NODES, D), jnp.float32),
    mesh=plsc.VectorSubcoreMesh(core_axis_name="c", subcore_axis_name="s"),
    scratch_types=[
        pltpu.VMEM((NCHQ, CH), jnp.int32),
        pltpu.VMEM((NCHQ, CH), jnp.int32),
        pltpu.VMEM((NB, CH, D), jnp.float32),
        pltpu.VMEM((8, D), jnp.float32),
        [pltpu.SemaphoreType.DMA] * NB,
        pltpu.VMEM_SHARED((N_ACC, D), jnp.float32),
    ],
)


BLK = 1000


def _tc_lin1_body(x_ref, w_ref, h0_ref, h1_ref, hs_ref, dinv_ref):
    deg = h0_ref[...] + h1_ref[...] + 1.0
    dinv = lax.rsqrt(deg)
    h = jnp.dot(x_ref[...], w_ref[...], preferred_element_type=jnp.float32)
    hs_ref[...] = h * dinv
    dinv_ref[...] = dinv


def _tc_lin1(x, w1, h0, h1):
    return pl.pallas_call(
        _tc_lin1_body,
        grid=(N_NODES // BLK,),
        in_specs=[
            pl.BlockSpec((BLK, D), lambda i: (i, 0)),
            pl.BlockSpec((D, D), lambda i: (0, 0)),
            pl.BlockSpec((BLK, 1), lambda i: (i, 0)),
            pl.BlockSpec((BLK, 1), lambda i: (i, 0)),
        ],
        out_specs=[
            pl.BlockSpec((BLK, D), lambda i: (i, 0)),
            pl.BlockSpec((BLK, 1), lambda i: (i, 0)),
        ],
        out_shape=[
            jax.ShapeDtypeStruct((N_NODES, D), jnp.float32),
            jax.ShapeDtypeStruct((N_NODES, 1), jnp.float32),
        ],
    )(x, w1, h0, h1)


def _tc_mid_body(p0_ref, p1_ref, hs_ref, dinv_ref, b_ref, w_ref, out_ref):
    dinv = dinv_ref[...]
    p = p0_ref[...].reshape(BLK, D) + p1_ref[...].reshape(BLK, D)
    t = (p + hs_ref[...]) * dinv + b_ref[...]
    r = jnp.maximum(t, 0.0)
    out_ref[...] = jnp.dot(
        r, w_ref[...], preferred_element_type=jnp.float32) * dinv


def _tc_mid(p, hs, dinv, b1, w2):
    return pl.pallas_call(
        _tc_mid_body,
        grid=(N_NODES // BLK,),
        in_specs=[
            pl.BlockSpec((1, BLK, D), lambda i: (0, i, 0)),
            pl.BlockSpec((1, BLK, D), lambda i: (1, i, 0)),
            pl.BlockSpec((BLK, D), lambda i: (i, 0)),
            pl.BlockSpec((BLK, 1), lambda i: (i, 0)),
            pl.BlockSpec((1, D), lambda i: (0, 0)),
            pl.BlockSpec((D, D), lambda i: (0, 0)),
        ],
        out_specs=pl.BlockSpec((BLK, D), lambda i: (i, 0)),
        out_shape=jax.ShapeDtypeStruct((N_NODES, D), jnp.float32),
    )(p, p, hs, dinv, b1, w2)


def _tc_head_body(q0_ref, q1_ref, hs_ref, dinv_ref, b_ref, batch_ref,
                  wf_ref, bf_ref, out_ref, sums_ref, cnts_ref):
    i = pl.program_id(0)

    @pl.when(i == 0)
    def _():
        sums_ref[...] = jnp.zeros_like(sums_ref)
        cnts_ref[...] = jnp.zeros_like(cnts_ref)

    q = q0_ref[...].reshape(BLK, D) + q1_ref[...].reshape(BLK, D)
    t = (q + hs_ref[...]) * dinv_ref[...] + b_ref[...]
    r = jnp.maximum(t, 0.0)

    gids = lax.broadcasted_iota(jnp.int32, (N_GRAPHS, BLK), 0)
    brow = batch_ref[...].reshape(1, BLK)
    mask = (gids == brow).astype(jnp.float32)
    sums_ref[...] += jnp.dot(mask, r, preferred_element_type=jnp.float32,
                             precision=lax.Precision.HIGHEST)
    cnts_ref[...] += jnp.broadcast_to(
        jnp.sum(mask, axis=1, keepdims=True), (N_GRAPHS, D))

    @pl.when(i == (N_NODES // BLK) - 1)
    def _():
        pooled = sums_ref[...] / jnp.maximum(cnts_ref[...], 1.0)
        out_ref[...] = jnp.dot(
            pooled, wf_ref[...], preferred_element_type=jnp.float32
        ) + bf_ref[...]


def _tc_head(q, hs2, dinv, b2, batch_rows, wf, bf):
    return pl.pallas_call(
        _tc_head_body,
        grid=(N_NODES // BLK,),
        in_specs=[
            pl.BlockSpec((1, BLK, D), lambda i: (0, i, 0)),
            pl.BlockSpec((1, BLK, D), lambda i: (1, i, 0)),
            pl.BlockSpec((BLK, D), lambda i: (i, 0)),
            pl.BlockSpec((BLK, 1), lambda i: (i, 0)),
            pl.BlockSpec((1, D), lambda i: (0, 0)),
            pl.BlockSpec((1, 1, BLK), lambda i: (i, 0, 0)),
            pl.BlockSpec((D, 1), lambda i: (0, 0)),
            pl.BlockSpec((1, 1), lambda i: (0, 0)),
        ],
        out_specs=pl.BlockSpec((N_GRAPHS, 1), lambda i: (0, 0)),
        out_shape=jax.ShapeDtypeStruct((N_GRAPHS, 1), jnp.float32),
        scratch_shapes=[
            pltpu.VMEM((N_GRAPHS, D), jnp.float32),
            pltpu.VMEM((N_GRAPHS, D), jnp.float32),
        ],
    )(q, q, hs2, dinv, b2, batch_rows, wf, bf)


@jax.jit
def kernel(x, edge_index, batch, W1, b1, W2, b2, Wf, bf):
    i32 = jnp.int32

    src3 = edge_index[0].astype(i32).reshape(NW, NQ, NCHQ, CH)
    dst3 = edge_index[1].astype(i32).reshape(NW, NQ, NCHQ, CH)
    batch_rows = batch.astype(i32).reshape(N_NODES // BLK, 1, BLK)

    hist = _sc_hist(dst3)

    hs1, dinv = _tc_lin1(x.astype(jnp.float32), W1,
                         hist[0, :N_NODES].reshape(N_NODES, 1),
                         hist[1, :N_NODES].reshape(N_NODES, 1))

    p = _sc_scatter(hs1, src3, dst3)
    hs2 = _tc_mid(p, hs1, dinv, b1.reshape(1, D), W2)

    q = _sc_scatter(hs2, src3, dst3)
    out = _tc_head(q, hs2, dinv, b2.reshape(1, D), batch_rows,
                   Wf, bf.reshape(1, 1))
    return out
